# baseline (speedup 1.0000x reference)
import jax
import jax.numpy as jnp
from jax.experimental import pallas as pl
from jax.experimental.pallas import tpu as pltpu

_BINS = 30
_K = 512


def _stats_body(a_ref, b_ref, o_ref):
    i = pl.program_id(0)
    a = a_ref[...]
    b = b_ref[...]
    mn_a = jnp.min(a, axis=0, keepdims=True)
    mx_a = jnp.max(a, axis=0, keepdims=True)
    mn_b = jnp.min(b, axis=0, keepdims=True)
    mx_b = jnp.max(b, axis=0, keepdims=True)
    cur = jnp.concatenate(
        [mn_a, mx_a, mn_b, mx_b, mn_a, mx_a, mn_b, mx_b], axis=0
    )

    @pl.when(i == 0)
    def _():
        o_ref[...] = cur

    @pl.when(i != 0)
    def _():
        prev = o_ref[...]
        comb_mn = jnp.minimum(prev, cur)
        comb_mx = jnp.maximum(prev, cur)
        sel = jax.lax.broadcasted_iota(jnp.int32, prev.shape, 0) % 2
        o_ref[...] = jnp.where(sel == 0, comb_mn, comb_mx)


def _hist_body(stats_ref, a_ref, b_ref, ca_ref, cb_ref, acc_ref):
    i = pl.program_id(0)

    @pl.when(i == 0)
    def _():
        ca_ref[...] = jnp.zeros_like(ca_ref)
        cb_ref[...] = jnp.zeros_like(cb_ref)

    def hist(x_ref, mn, mx, c_ref):
        rng = mx - mn
        safe = jnp.where(rng == 0.0, 1.0, rng)
        nrows = x_ref.shape[0]
        for chunk in range(nrows // 512):
            acc_ref[...] = jnp.zeros_like(acc_ref)

            def slab_body(s, carry):
                x = x_ref[pl.ds(chunk * 512 + s * 8, 8), :]
                norm = (x - mn) / safe
                t = jnp.clip(
                    jnp.floor(norm * float(_BINS)), 0.0, float(_BINS - 1)
                ).astype(jnp.int32)
                g = t >> 2
                sh = (t & 3) << 3
                one = jnp.left_shift(jnp.ones_like(t), sh)
                zero = jnp.zeros_like(t)
                for gi in range(8):
                    acc_ref[8 * gi:8 * gi + 8, :] += jnp.where(g == gi, one, zero)
                return carry

            jax.lax.fori_loop(0, 64, slab_body, 0)
            for k in range(_BINS):
                gi, j = k >> 2, k & 3
                cnt = (acc_ref[8 * gi:8 * gi + 8, :] >> (8 * j)) & 255
                c_ref[k:k + 1, :] += jnp.sum(
                    cnt, axis=0, keepdims=True).astype(jnp.float32)

    hist(a_ref, stats_ref[0:1, :], stats_ref[1:2, :], ca_ref)
    hist(b_ref, stats_ref[2:3, :], stats_ref[3:4, :], cb_ref)


def _tree_sum_rows(t):
    s = ((t[0:8, :] + t[8:16, :]) + t[16:24, :]) + t[24:32, :]
    while s.shape[0] > 1:
        h = s.shape[0] // 2
        s = s[:h, :] + s[h:, :]
    return s


def _select_body(stats_ref, ca_ref, cb_ref, pa_ref, pb_ref):
    f = ca_ref.shape[1]

    def build(c, rng, p_ref):
        s = jnp.sum(c, axis=0, keepdims=True)
        p = c / s
        q = p + 1e-12
        t = p * jnp.log(q)
        ent = -_tree_sum_rows(t)
        ent = jnp.where(rng == 0.0, 0.0, ent)
        e_row = jnp.broadcast_to(ent, (f, f))
        e_col = e_row.T
        r_iota = jax.lax.broadcasted_iota(jnp.int32, (f, f), 0)
        c_iota = jax.lax.broadcasted_iota(jnp.int32, (f, f), 1)
        beats = (e_col > e_row) | ((e_col == e_row) & (r_iota < c_iota))
        rank = jnp.sum(beats.astype(jnp.float32), axis=0, keepdims=True)
        rank_col = jnp.broadcast_to(rank, (f, f)).T[:, :_K]
        p_iota = jax.lax.broadcasted_iota(jnp.int32, (f, _K), 1)
        p_ref[...] = (rank_col.astype(jnp.int32) == p_iota).astype(jnp.float32)

    rng_a = stats_ref[1:2, :] - stats_ref[0:1, :]
    rng_b = stats_ref[3:4, :] - stats_ref[2:3, :]
    build(ca_ref[...], rng_a, pa_ref)
    build(cb_ref[...], rng_b, pb_ref)


def _gather_body(pa_ref, pb_ref, a_ref, b_ref, o_ref):
    dn = (((1,), (0,)), ((), ()))

    def sel(x, p_bf16):
        hi = x.astype(jnp.bfloat16)
        r1 = x - hi.astype(jnp.float32)
        mid = r1.astype(jnp.bfloat16)
        lo = (r1 - mid.astype(jnp.float32)).astype(jnp.bfloat16)
        acc = jax.lax.dot_general(
            hi, p_bf16, dn, preferred_element_type=jnp.float32)
        acc = acc + jax.lax.dot_general(
            mid, p_bf16, dn, preferred_element_type=jnp.float32)
        return acc + jax.lax.dot_general(
            lo, p_bf16, dn, preferred_element_type=jnp.float32)

    o_ref[:, 0:_K] = sel(a_ref[...], pa_ref[...].astype(jnp.bfloat16))
    o_ref[:, _K:2 * _K] = sel(b_ref[...], pb_ref[...].astype(jnp.bfloat16))


def kernel(a, b):
    import kernel_sc
    return kernel_sc.kernel_sc(a, b)
    n, f = a.shape
    r1 = 2048
    g1 = n // r1

    row_spec1 = pl.BlockSpec((r1, f), lambda i: (i, 0))
    const8 = pl.BlockSpec((8, f), lambda i: (0, 0))
    const32 = pl.BlockSpec((32, f), lambda i: (0, 0))

    stats = pl.pallas_call(
        _stats_body,
        grid=(g1,),
        in_specs=[row_spec1, row_spec1],
        out_specs=const8,
        out_shape=jax.ShapeDtypeStruct((8, f), jnp.float32),
        compiler_params=pltpu.CompilerParams(
            dimension_semantics=("arbitrary",)),
    )(a, b)

    ca, cb = pl.pallas_call(
        _hist_body,
        grid=(g1,),
        in_specs=[const8, row_spec1, row_spec1],
        out_specs=[const32, const32],
        out_shape=[jax.ShapeDtypeStruct((32, f), jnp.float32),
                   jax.ShapeDtypeStruct((32, f), jnp.float32)],
        scratch_shapes=[pltpu.VMEM((64, f), jnp.int32)],
        compiler_params=pltpu.CompilerParams(
            dimension_semantics=("arbitrary",)),
    )(stats, a, b)

    pa, pb = pl.pallas_call(
        _select_body,
        in_specs=[pl.BlockSpec((8, f), lambda: (0, 0)),
                  pl.BlockSpec((32, f), lambda: (0, 0)),
                  pl.BlockSpec((32, f), lambda: (0, 0))],
        out_specs=[pl.BlockSpec((f, _K), lambda: (0, 0)),
                   pl.BlockSpec((f, _K), lambda: (0, 0))],
        out_shape=[jax.ShapeDtypeStruct((f, _K), jnp.float32),
                   jax.ShapeDtypeStruct((f, _K), jnp.float32)],
    )(stats, ca, cb)

    r2 = 1024
    g2 = n // r2
    row_spec2 = pl.BlockSpec((r2, f), lambda i: (i, 0))
    pk_spec = pl.BlockSpec((f, _K), lambda i: (0, 0))
    out = pl.pallas_call(
        _gather_body,
        grid=(g2,),
        in_specs=[pk_spec, pk_spec, row_spec2, row_spec2],
        out_specs=row_spec2,
        out_shape=jax.ShapeDtypeStruct((n, 2 * _K), jnp.float32),
        compiler_params=pltpu.CompilerParams(
            dimension_semantics=("parallel",)),
    )(pa, pb, a, b)
    return out


# speedup vs baseline: 2.4727x; 2.4727x over previous
import jax
import jax.numpy as jnp
from jax.experimental import pallas as pl
from jax.experimental.pallas import tpu as pltpu

_BINS = 30
_K = 512


def _stats_body(a_ref, b_ref, o_ref):
    i = pl.program_id(0)
    a = a_ref[...]
    b = b_ref[...]
    mn_a = jnp.min(a, axis=0, keepdims=True)
    mx_a = jnp.max(a, axis=0, keepdims=True)
    mn_b = jnp.min(b, axis=0, keepdims=True)
    mx_b = jnp.max(b, axis=0, keepdims=True)
    cur = jnp.concatenate(
        [mn_a, mx_a, mn_b, mx_b, mn_a, mx_a, mn_b, mx_b], axis=0
    )

    @pl.when(i == 0)
    def _():
        o_ref[...] = cur

    @pl.when(i != 0)
    def _():
        prev = o_ref[...]
        comb_mn = jnp.minimum(prev, cur)
        comb_mx = jnp.maximum(prev, cur)
        sel = jax.lax.broadcasted_iota(jnp.int32, prev.shape, 0) % 2
        o_ref[...] = jnp.where(sel == 0, comb_mn, comb_mx)


def _hist_body(stats_ref, a_ref, b_ref, ca_ref, cb_ref, acc_ref):
    i = pl.program_id(0)

    @pl.when(i == 0)
    def _():
        ca_ref[...] = jnp.zeros_like(ca_ref)
        cb_ref[...] = jnp.zeros_like(cb_ref)

    def hist(x_ref, mn, mx, c_ref):
        rng = mx - mn
        safe = jnp.where(rng == 0.0, 1.0, rng)
        nrows = x_ref.shape[0]
        for chunk in range(nrows // 512):
            acc_ref[...] = jnp.zeros_like(acc_ref)

            def slab_body(s4, carry):
                conts = [None] * 8
                for d in range(4):
                    x = x_ref[pl.ds(chunk * 512 + (s4 * 4 + d) * 8, 8), :]
                    norm = (x - mn) / safe
                    t = jnp.clip(
                        jnp.floor(norm * float(_BINS)), 0.0, float(_BINS - 1)
                    ).astype(jnp.int32)
                    g = t >> 2
                    sh = (t & 3) << 3
                    one = jnp.left_shift(jnp.ones_like(t), sh)
                    zero = jnp.zeros_like(t)
                    for gi in range(8):
                        w = jnp.where(g == gi, one, zero)
                        conts[gi] = w if conts[gi] is None else conts[gi] + w
                for gi in range(8):
                    acc_ref[8 * gi:8 * gi + 8, :] += conts[gi]
                return carry

            jax.lax.fori_loop(0, 16, slab_body, 0)
            for k in range(_BINS):
                gi, j = k >> 2, k & 3
                cnt = (acc_ref[8 * gi:8 * gi + 8, :] >> (8 * j)) & 255
                c_ref[k:k + 1, :] += jnp.sum(
                    cnt, axis=0, keepdims=True).astype(jnp.float32)

    hist(a_ref, stats_ref[0:1, :], stats_ref[1:2, :], ca_ref)
    hist(b_ref, stats_ref[2:3, :], stats_ref[3:4, :], cb_ref)


def _tree_sum_rows(t):
    s = ((t[0:8, :] + t[8:16, :]) + t[16:24, :]) + t[24:32, :]
    while s.shape[0] > 1:
        h = s.shape[0] // 2
        s = s[:h, :] + s[h:, :]
    return s


def _select_body(stats_ref, ca_ref, cb_ref, pa_ref, pb_ref):
    f = ca_ref.shape[1]

    def build(c, rng, p_ref):
        s = jnp.sum(c, axis=0, keepdims=True)
        p = c / s
        q = p + 1e-12
        t = p * jnp.log(q)
        ent = -_tree_sum_rows(t)
        ent = jnp.where(rng == 0.0, 0.0, ent)
        e_row = jnp.broadcast_to(ent, (f, f))
        e_col = e_row.T
        r_iota = jax.lax.broadcasted_iota(jnp.int32, (f, f), 0)
        c_iota = jax.lax.broadcasted_iota(jnp.int32, (f, f), 1)
        beats = (e_col > e_row) | ((e_col == e_row) & (r_iota < c_iota))
        rank = jnp.sum(beats.astype(jnp.float32), axis=0, keepdims=True)
        rank_col = jnp.broadcast_to(rank, (f, f)).T[:, :_K]
        p_iota = jax.lax.broadcasted_iota(jnp.int32, (f, _K), 1)
        p_ref[...] = (rank_col.astype(jnp.int32) == p_iota).astype(jnp.float32)

    rng_a = stats_ref[1:2, :] - stats_ref[0:1, :]
    rng_b = stats_ref[3:4, :] - stats_ref[2:3, :]
    build(ca_ref[...], rng_a, pa_ref)
    build(cb_ref[...], rng_b, pb_ref)


def _gather_body(pa_ref, pb_ref, a_ref, b_ref, o_ref):
    dn = (((1,), (0,)), ((), ()))

    def sel(x, p_bf16):
        hi = x.astype(jnp.bfloat16)
        r1 = x - hi.astype(jnp.float32)
        mid = r1.astype(jnp.bfloat16)
        lo = (r1 - mid.astype(jnp.float32)).astype(jnp.bfloat16)
        acc = jax.lax.dot_general(
            hi, p_bf16, dn, preferred_element_type=jnp.float32)
        acc = acc + jax.lax.dot_general(
            mid, p_bf16, dn, preferred_element_type=jnp.float32)
        return acc + jax.lax.dot_general(
            lo, p_bf16, dn, preferred_element_type=jnp.float32)

    o_ref[:, 0:_K] = sel(a_ref[...], pa_ref[...].astype(jnp.bfloat16))
    o_ref[:, _K:2 * _K] = sel(b_ref[...], pb_ref[...].astype(jnp.bfloat16))


def kernel(a, b):
    n, f = a.shape
    r1 = 2048
    g1 = n // r1

    row_spec1 = pl.BlockSpec((r1, f), lambda i: (i, 0))
    const8 = pl.BlockSpec((8, f), lambda i: (0, 0))
    const32 = pl.BlockSpec((32, f), lambda i: (0, 0))

    stats = pl.pallas_call(
        _stats_body,
        grid=(g1,),
        in_specs=[row_spec1, row_spec1],
        out_specs=const8,
        out_shape=jax.ShapeDtypeStruct((8, f), jnp.float32),
        compiler_params=pltpu.CompilerParams(
            dimension_semantics=("arbitrary",)),
    )(a, b)

    ca, cb = pl.pallas_call(
        _hist_body,
        grid=(g1,),
        in_specs=[const8, row_spec1, row_spec1],
        out_specs=[const32, const32],
        out_shape=[jax.ShapeDtypeStruct((32, f), jnp.float32),
                   jax.ShapeDtypeStruct((32, f), jnp.float32)],
        scratch_shapes=[pltpu.VMEM((64, f), jnp.int32)],
        compiler_params=pltpu.CompilerParams(
            dimension_semantics=("arbitrary",)),
    )(stats, a, b)

    pa, pb = pl.pallas_call(
        _select_body,
        in_specs=[pl.BlockSpec((8, f), lambda: (0, 0)),
                  pl.BlockSpec((32, f), lambda: (0, 0)),
                  pl.BlockSpec((32, f), lambda: (0, 0))],
        out_specs=[pl.BlockSpec((f, _K), lambda: (0, 0)),
                   pl.BlockSpec((f, _K), lambda: (0, 0))],
        out_shape=[jax.ShapeDtypeStruct((f, _K), jnp.float32),
                   jax.ShapeDtypeStruct((f, _K), jnp.float32)],
    )(stats, ca, cb)

    r2 = 1024
    g2 = n // r2
    row_spec2 = pl.BlockSpec((r2, f), lambda i: (i, 0))
    pk_spec = pl.BlockSpec((f, _K), lambda i: (0, 0))
    out = pl.pallas_call(
        _gather_body,
        grid=(g2,),
        in_specs=[pk_spec, pk_spec, row_spec2, row_spec2],
        out_specs=row_spec2,
        out_shape=jax.ShapeDtypeStruct((n, 2 * _K), jnp.float32),
        compiler_params=pltpu.CompilerParams(
            dimension_semantics=("parallel",)),
    )(pa, pb, a, b)
    return out
